# Initial kernel scaffold; baseline (speedup 1.0000x reference)
import jax
import jax.numpy as jnp
from jax.experimental import pallas as pl


def kernel(x, edge_index, W1, b1, W2, b2):
    raise NotImplementedError("write your pallas kernel here")


# speedup vs baseline: 9.2060x; 9.2060x over previous
import functools

import jax
import jax.numpy as jnp
from jax import lax
from jax.experimental import pallas as pl
from jax.experimental.pallas import tpu as pltpu
from jax.experimental.pallas import tpu_sc as plsc

N_NODES = 10000
D = 128
N_EDGES = 320000

NC = 2
NS = 16
NW = NC * NS
CHUNK = 128
CPW = 80
E_PAD = NW * CPW * CHUNK
ROWS_P = 10240
RPT = ROWS_P // NS
HIST_W = D

_mesh = plsc.VectorSubcoreMesh(core_axis_name="c", subcore_axis_name="s")


def _zero_fill(buf, width):
  def body(i, _):
    for k in range(width // 16):
      buf[i, pl.ds(k * 16, 16)] = jnp.zeros((16,), jnp.float32)
    return 0
  lax.fori_loop(0, CHUNK, body, 0)


def _zero_acc_slice(zeros_v, acc_sh, s):
  base = s * RPT
  off = 0
  while off < RPT:
    n = min(CHUNK, RPT - off)
    pltpu.sync_copy(zeros_v.at[pl.ds(0, n)], acc_sh.at[pl.ds(base + off, n)])
    off += n


@functools.partial(
    pl.kernel,
    out_type=jax.ShapeDtypeStruct((NC, ROWS_P, HIST_W), jnp.float32),
    mesh=_mesh,
    scratch_types=[
        pltpu.VMEM((CPW, CHUNK), jnp.int32),
        pltpu.VMEM((CHUNK, HIST_W), jnp.float32),
        pltpu.VMEM((CHUNK, HIST_W), jnp.float32),
        pltpu.VMEM_SHARED((ROWS_P, HIST_W), jnp.float32),
    ],
)
def _hist(dst_hbm, out_hbm, idx_v, ones_v, zeros_v, acc_sh):
  c = lax.axis_index("c")
  s = lax.axis_index("s")
  w = s * NC + c

  def fill(i, _):
    for k in range(HIST_W // 16):
      ones_v[i, pl.ds(k * 16, 16)] = jnp.ones((16,), jnp.float32)
      zeros_v[i, pl.ds(k * 16, 16)] = jnp.zeros((16,), jnp.float32)
    return 0
  lax.fori_loop(0, CHUNK, fill, 0)

  _zero_acc_slice(zeros_v, acc_sh, s)
  plsc.subcore_barrier()

  pltpu.sync_copy(dst_hbm.at[w], idx_v)

  def body(j, _):
    pltpu.sync_copy(ones_v, acc_sh.at[idx_v.at[j]], add=True)
    return 0
  lax.fori_loop(0, CPW, body, 0)

  plsc.subcore_barrier()
  base = s * RPT
  pltpu.sync_copy(acc_sh.at[pl.ds(base, RPT)], out_hbm.at[c, pl.ds(base, RPT)])


@functools.partial(
    pl.kernel,
    out_type=jax.ShapeDtypeStruct((NC, ROWS_P, D), jnp.float32),
    mesh=_mesh,
    scratch_types=[
        pltpu.VMEM((CPW, CHUNK), jnp.int32),
        pltpu.VMEM((CPW, CHUNK), jnp.int32),
        pltpu.VMEM((CHUNK, D), jnp.float32),
        pltpu.VMEM_SHARED((ROWS_P, D), jnp.float32),
    ],
)
def _agg(table_hbm, src_hbm, dst_hbm, out_hbm, src_v, dst_v, buf_v, acc_sh):
  c = lax.axis_index("c")
  s = lax.axis_index("s")
  w = s * NC + c

  _zero_fill(buf_v, D)
  _zero_acc_slice(buf_v, acc_sh, s)
  plsc.subcore_barrier()

  pltpu.sync_copy(src_hbm.at[w], src_v)
  pltpu.sync_copy(dst_hbm.at[w], dst_v)

  def body(j, _):
    pltpu.sync_copy(table_hbm.at[src_v.at[j]], buf_v)
    pltpu.sync_copy(buf_v, acc_sh.at[dst_v.at[j]], add=True)
    return 0
  lax.fori_loop(0, CPW, body, 0)

  plsc.subcore_barrier()
  base = s * RPT
  pltpu.sync_copy(acc_sh.at[pl.ds(base, RPT)], out_hbm.at[c, pl.ds(base, RPT)])


_R = 1000


def _dis(h0, h1):
  return lax.rsqrt(1.0 + h0[...] + h1[...])


def _scale_matmul_body(x_ref, h0_ref, h1_ref, w_ref, o_ref):
  dis = _dis(h0_ref, h1_ref)
  o_ref[...] = jnp.dot(x_ref[...] * dis, w_ref[...],
                       preferred_element_type=jnp.float32)


def _combine_matmul_body(a0_ref, a1_ref, y_ref, h0_ref, h1_ref, b_ref, w_ref,
                         o_ref):
  dis = _dis(h0_ref, h1_ref)
  t = dis * (a0_ref[...] + a1_ref[...] + y_ref[...]) + b_ref[...]
  o_ref[...] = jnp.dot(dis * jnp.maximum(t, 0.0), w_ref[...],
                       preferred_element_type=jnp.float32)


def _combine_body(a0_ref, a1_ref, y_ref, h0_ref, h1_ref, b_ref, o_ref):
  dis = _dis(h0_ref, h1_ref)
  o_ref[...] = dis * (a0_ref[...] + a1_ref[...] + y_ref[...]) + b_ref[...]


def _row_spec():
  return pl.BlockSpec((_R, D), lambda i: (i, 0))


def _col_spec():
  return pl.BlockSpec((_R, 1), lambda i: (i, 0))


def _full_spec(r):
  return pl.BlockSpec((r, D), lambda i: (0, 0))


_GRID = N_NODES // _R

_scale_matmul = pl.pallas_call(
    _scale_matmul_body,
    grid=(_GRID,),
    in_specs=[_row_spec(), _col_spec(), _col_spec(), _full_spec(D)],
    out_specs=_row_spec(),
    out_shape=jax.ShapeDtypeStruct((N_NODES, D), jnp.float32),
)

_combine_matmul = pl.pallas_call(
    _combine_matmul_body,
    grid=(_GRID,),
    in_specs=[_row_spec(), _row_spec(), _row_spec(), _col_spec(), _col_spec(),
              _full_spec(1), _full_spec(D)],
    out_specs=_row_spec(),
    out_shape=jax.ShapeDtypeStruct((N_NODES, D), jnp.float32),
)

_combine = pl.pallas_call(
    _combine_body,
    grid=(_GRID,),
    in_specs=[_row_spec(), _row_spec(), _row_spec(), _col_spec(), _col_spec(),
              _full_spec(1)],
    out_specs=_row_spec(),
    out_shape=jax.ShapeDtypeStruct((N_NODES, D), jnp.float32),
)


def kernel(x, edge_index, W1, b1, W2, b2):
  src = edge_index[0].astype(jnp.int32)
  dst = edge_index[1].astype(jnp.int32)
  pad = jnp.full((E_PAD - N_EDGES,), N_NODES, jnp.int32)
  srcp = jnp.concatenate([src, pad]).reshape(NW, CPW, CHUNK)
  dstp = jnp.concatenate([dst, pad]).reshape(NW, CPW, CHUNK)

  hist = _hist(dstp)
  h0 = hist[0, :N_NODES, 0:1]
  h1 = hist[1, :N_NODES, 0:1]

  zpad = jnp.zeros((ROWS_P - N_NODES, D), jnp.float32)

  y1 = _scale_matmul(x, h0, h1, W1)
  acc1 = _agg(jnp.concatenate([y1, zpad]), srcp, dstp)
  y2 = _combine_matmul(acc1[0, :N_NODES], acc1[1, :N_NODES], y1, h0, h1,
                       b1.reshape(1, D), W2)
  acc2 = _agg(jnp.concatenate([y2, zpad]), srcp, dstp)
  return _combine(acc2[0, :N_NODES], acc2[1, :N_NODES], y2, h0, h1,
                  b2.reshape(1, D))


# speedup vs baseline: 9.8541x; 1.0704x over previous
import functools

import jax
import jax.numpy as jnp
from jax import lax
from jax.experimental import pallas as pl
from jax.experimental.pallas import tpu as pltpu
from jax.experimental.pallas import tpu_sc as plsc

N_NODES = 10000
D = 128
N_EDGES = 320000

NC = 2
NS = 16
NW = NC * NS
CHUNK = 128
CPW = 80
E_PAD = NW * CPW * CHUNK
ROWS_P = 10240
RPT = ROWS_P // NS
HIST_W = D

_mesh = plsc.VectorSubcoreMesh(core_axis_name="c", subcore_axis_name="s")


def _zero_fill(buf, width):
  def body(i, _):
    for k in range(width // 16):
      buf[i, pl.ds(k * 16, 16)] = jnp.zeros((16,), jnp.float32)
    return 0
  lax.fori_loop(0, CHUNK, body, 0)


def _zero_acc_slice(zeros_v, acc_sh, s):
  base = s * RPT
  off = 0
  while off < RPT:
    n = min(CHUNK, RPT - off)
    pltpu.sync_copy(zeros_v.at[pl.ds(0, n)], acc_sh.at[pl.ds(base + off, n)])
    off += n


@functools.partial(
    pl.kernel,
    out_type=jax.ShapeDtypeStruct((NC, ROWS_P, HIST_W), jnp.float32),
    mesh=_mesh,
    scratch_types=[
        pltpu.VMEM((CPW, CHUNK), jnp.int32),
        pltpu.VMEM((CHUNK, HIST_W), jnp.float32),
        pltpu.VMEM((CHUNK, HIST_W), jnp.float32),
        pltpu.VMEM_SHARED((ROWS_P, HIST_W), jnp.float32),
    ],
)
def _hist(dst_hbm, out_hbm, idx_v, ones_v, zeros_v, acc_sh):
  c = lax.axis_index("c")
  s = lax.axis_index("s")
  w = s * NC + c

  def fill(i, _):
    for k in range(HIST_W // 16):
      ones_v[i, pl.ds(k * 16, 16)] = jnp.ones((16,), jnp.float32)
      zeros_v[i, pl.ds(k * 16, 16)] = jnp.zeros((16,), jnp.float32)
    return 0
  lax.fori_loop(0, CHUNK, fill, 0)

  _zero_acc_slice(zeros_v, acc_sh, s)
  plsc.subcore_barrier()

  pltpu.sync_copy(dst_hbm.at[w], idx_v)

  def body(j, _):
    pltpu.sync_copy(ones_v, acc_sh.at[idx_v.at[j]], add=True)
    return 0
  lax.fori_loop(0, CPW, body, 0)

  plsc.subcore_barrier()
  base = s * RPT
  pltpu.sync_copy(acc_sh.at[pl.ds(base, RPT)], out_hbm.at[c, pl.ds(base, RPT)])


NBUF = 2
SEG = 16


@functools.partial(
    pl.kernel,
    out_type=jax.ShapeDtypeStruct((NC, ROWS_P, D), jnp.float32),
    mesh=_mesh,
    scratch_types=[
        pltpu.VMEM((SEG, CHUNK), jnp.int32),
        pltpu.VMEM((SEG, CHUNK), jnp.int32),
        pltpu.VMEM((NBUF, CHUNK, D), jnp.float32),
        pltpu.VMEM_SHARED((ROWS_P, D), jnp.float32),
        pltpu.SemaphoreType.DMA,
        pltpu.SemaphoreType.DMA,
    ],
)
def _agg(table_hbm, src_hbm, dst_hbm, out_hbm, src_v, dst_v, buf_v, acc_sh,
         sem0, sem1):
  sems = (sem0, sem1)
  c = lax.axis_index("c")
  s = lax.axis_index("s")
  w = s * NC + c

  _zero_fill(buf_v.at[0], D)
  _zero_acc_slice(buf_v.at[0], acc_sh, s)
  plsc.subcore_barrier()

  def seg_body(t, _):
    pltpu.sync_copy(src_hbm.at[w, pl.ds(t * SEG, SEG)], src_v)
    pltpu.sync_copy(dst_hbm.at[w, pl.ds(t * SEG, SEG)], dst_v)
    pltpu.async_copy(table_hbm.at[src_v.at[0]], buf_v.at[0], sems[0])

    def body(g, _):
      for b in range(NBUF):
        j = g * NBUF + b
        pltpu.make_async_copy(table_hbm.at[src_v.at[j]], buf_v.at[b],
                              sems[b]).wait()
        jn = j + 1

        @pl.when(jn < SEG)
        def _():
          bn = (b + 1) % NBUF
          pltpu.async_copy(table_hbm.at[src_v.at[jn]], buf_v.at[bn], sems[bn])

        pltpu.sync_copy(buf_v.at[b], acc_sh.at[dst_v.at[j]], add=True)
      return 0
    lax.fori_loop(0, SEG // NBUF, body, 0)
    return 0
  lax.fori_loop(0, CPW // SEG, seg_body, 0)

  plsc.subcore_barrier()
  base = s * RPT
  pltpu.sync_copy(acc_sh.at[pl.ds(base, RPT)], out_hbm.at[c, pl.ds(base, RPT)])


_R = 1000


def _dis(h0, h1):
  return lax.rsqrt(1.0 + h0[...] + h1[...])


def _scale_matmul_body(x_ref, h0_ref, h1_ref, w_ref, o_ref):
  dis = _dis(h0_ref, h1_ref)
  o_ref[...] = jnp.dot(x_ref[...] * dis, w_ref[...],
                       preferred_element_type=jnp.float32)


def _combine_matmul_body(a0_ref, a1_ref, y_ref, h0_ref, h1_ref, b_ref, w_ref,
                         o_ref):
  dis = _dis(h0_ref, h1_ref)
  t = dis * (a0_ref[...] + a1_ref[...] + y_ref[...]) + b_ref[...]
  o_ref[...] = jnp.dot(dis * jnp.maximum(t, 0.0), w_ref[...],
                       preferred_element_type=jnp.float32)


def _combine_body(a0_ref, a1_ref, y_ref, h0_ref, h1_ref, b_ref, o_ref):
  dis = _dis(h0_ref, h1_ref)
  o_ref[...] = dis * (a0_ref[...] + a1_ref[...] + y_ref[...]) + b_ref[...]


def _row_spec():
  return pl.BlockSpec((_R, D), lambda i: (i, 0))


def _col_spec():
  return pl.BlockSpec((_R, 1), lambda i: (i, 0))


def _full_spec(r):
  return pl.BlockSpec((r, D), lambda i: (0, 0))


_GRID = N_NODES // _R

_scale_matmul = pl.pallas_call(
    _scale_matmul_body,
    grid=(_GRID,),
    in_specs=[_row_spec(), _col_spec(), _col_spec(), _full_spec(D)],
    out_specs=_row_spec(),
    out_shape=jax.ShapeDtypeStruct((N_NODES, D), jnp.float32),
)

_combine_matmul = pl.pallas_call(
    _combine_matmul_body,
    grid=(_GRID,),
    in_specs=[_row_spec(), _row_spec(), _row_spec(), _col_spec(), _col_spec(),
              _full_spec(1), _full_spec(D)],
    out_specs=_row_spec(),
    out_shape=jax.ShapeDtypeStruct((N_NODES, D), jnp.float32),
)

_combine = pl.pallas_call(
    _combine_body,
    grid=(_GRID,),
    in_specs=[_row_spec(), _row_spec(), _row_spec(), _col_spec(), _col_spec(),
              _full_spec(1)],
    out_specs=_row_spec(),
    out_shape=jax.ShapeDtypeStruct((N_NODES, D), jnp.float32),
)


def kernel(x, edge_index, W1, b1, W2, b2):
  src = edge_index[0].astype(jnp.int32)
  dst = edge_index[1].astype(jnp.int32)
  pad = jnp.full((E_PAD - N_EDGES,), N_NODES, jnp.int32)
  srcp = jnp.concatenate([src, pad]).reshape(NW, CPW, CHUNK)
  dstp = jnp.concatenate([dst, pad]).reshape(NW, CPW, CHUNK)

  hist = _hist(dstp)
  h0 = hist[0, :N_NODES, 0:1]
  h1 = hist[1, :N_NODES, 0:1]

  zpad = jnp.zeros((ROWS_P - N_NODES, D), jnp.float32)

  y1 = _scale_matmul(x, h0, h1, W1)
  acc1 = _agg(jnp.concatenate([y1, zpad]), srcp, dstp)
  y2 = _combine_matmul(acc1[0, :N_NODES], acc1[1, :N_NODES], y1, h0, h1,
                       b1.reshape(1, D), W2)
  acc2 = _agg(jnp.concatenate([y2, zpad]), srcp, dstp)
  return _combine(acc2[0, :N_NODES], acc2[1, :N_NODES], y2, h0, h1,
                  b2.reshape(1, D))


# speedup vs baseline: 23.3556x; 2.3701x over previous
import functools

import jax
import jax.numpy as jnp
from jax import lax
from jax.experimental import pallas as pl
from jax.experimental.pallas import tpu as pltpu
from jax.experimental.pallas import tpu_sc as plsc

N_NODES = 10000
D = 128
N_EDGES = 320000

NC = 2
NS = 16
NW = NC * NS
CHUNK = 128
CPW = 80
E_PAD = NW * CPW * CHUNK
ROWS_P = 10240
RPT = ROWS_P // NS
HIST_W = D

_mesh = plsc.VectorSubcoreMesh(core_axis_name="c", subcore_axis_name="s")


def _zero_fill(buf, width):
  def body(i, _):
    for k in range(width // 16):
      buf[i, pl.ds(k * 16, 16)] = jnp.zeros((16,), jnp.float32)
    return 0
  lax.fori_loop(0, CHUNK, body, 0)


def _zero_acc_slice(zeros_v, acc_sh, s):
  base = s * RPT
  off = 0
  while off < RPT:
    n = min(CHUNK, RPT - off)
    pltpu.sync_copy(zeros_v.at[pl.ds(0, n)], acc_sh.at[pl.ds(base + off, n)])
    off += n


@functools.partial(
    pl.kernel,
    out_type=jax.ShapeDtypeStruct((NC, ROWS_P, HIST_W), jnp.float32),
    mesh=_mesh,
    scratch_types=[
        pltpu.VMEM((CPW, CHUNK), jnp.int32),
        pltpu.VMEM((CHUNK, HIST_W), jnp.float32),
        pltpu.VMEM((CHUNK, HIST_W), jnp.float32),
        pltpu.VMEM_SHARED((ROWS_P, HIST_W), jnp.float32),
    ],
)
def _hist(dst_hbm, out_hbm, idx_v, ones_v, zeros_v, acc_sh):
  c = lax.axis_index("c")
  s = lax.axis_index("s")
  w = s * NC + c

  def fill(i, _):
    for k in range(HIST_W // 16):
      ones_v[i, pl.ds(k * 16, 16)] = jnp.ones((16,), jnp.float32)
      zeros_v[i, pl.ds(k * 16, 16)] = jnp.zeros((16,), jnp.float32)
    return 0
  lax.fori_loop(0, CHUNK, fill, 0)

  _zero_acc_slice(zeros_v, acc_sh, s)
  plsc.subcore_barrier()

  pltpu.sync_copy(dst_hbm.at[w], idx_v)

  def body(j, _):
    pltpu.sync_copy(ones_v, acc_sh.at[idx_v.at[j]], add=True)
    return 0
  lax.fori_loop(0, CPW, body, 0)

  plsc.subcore_barrier()
  base = s * RPT
  pltpu.sync_copy(acc_sh.at[pl.ds(base, RPT)], out_hbm.at[c, pl.ds(base, RPT)])


NBUF = 2
SEG = 16


@functools.partial(
    pl.kernel,
    out_type=jax.ShapeDtypeStruct((NC, ROWS_P, D), jnp.float32),
    mesh=_mesh,
    scratch_types=[
        pltpu.VMEM((SEG, CHUNK), jnp.int32),
        pltpu.VMEM((SEG, CHUNK), jnp.int32),
        pltpu.VMEM((NBUF, CHUNK, D), jnp.float32),
        pltpu.VMEM_SHARED((ROWS_P, D), jnp.float32),
        pltpu.SemaphoreType.DMA,
        pltpu.SemaphoreType.DMA,
    ],
)
def _agg(table_hbm, src_hbm, dst_hbm, out_hbm, src_v, dst_v, buf_v, acc_sh,
         sem0, sem1):
  sems = (sem0, sem1)
  c = lax.axis_index("c")
  s = lax.axis_index("s")
  w = s * NC + c

  _zero_fill(buf_v.at[0], D)
  _zero_acc_slice(buf_v.at[0], acc_sh, s)
  plsc.subcore_barrier()

  def seg_body(t, _):
    pltpu.sync_copy(src_hbm.at[w, pl.ds(t * SEG, SEG)], src_v)
    pltpu.sync_copy(dst_hbm.at[w, pl.ds(t * SEG, SEG)], dst_v)
    pltpu.async_copy(table_hbm.at[src_v.at[0]], buf_v.at[0], sems[0])

    def body(g, _):
      for b in range(NBUF):
        j = g * NBUF + b
        pltpu.make_async_copy(table_hbm.at[src_v.at[j]], buf_v.at[b],
                              sems[b]).wait()
        jn = j + 1

        @pl.when(jn < SEG)
        def _():
          bn = (b + 1) % NBUF
          pltpu.async_copy(table_hbm.at[src_v.at[jn]], buf_v.at[bn], sems[bn])

        pltpu.sync_copy(buf_v.at[b], acc_sh.at[dst_v.at[j]], add=True)
      return 0
    lax.fori_loop(0, SEG // NBUF, body, 0)
    return 0
  lax.fori_loop(0, CPW // SEG, seg_body, 0)

  plsc.subcore_barrier()
  base = s * RPT
  pltpu.sync_copy(acc_sh.at[pl.ds(base, RPT)], out_hbm.at[c, pl.ds(base, RPT)])


_R = 1000


def _dis(h0, h1):
  return lax.rsqrt(1.0 + h0[...] + h1[...])


def _scale_matmul_body(x_ref, h0_ref, h1_ref, w_ref, o_ref):
  dis = _dis(h0_ref, h1_ref)
  o_ref[...] = jnp.dot(x_ref[...] * dis, w_ref[...],
                       preferred_element_type=jnp.float32)


def _combine_matmul_body(a0_ref, a1_ref, y_ref, h0_ref, h1_ref, b_ref, w_ref,
                         o_ref):
  dis = _dis(h0_ref, h1_ref)
  t = dis * (a0_ref[...] + a1_ref[...] + y_ref[...]) + b_ref[...]
  o_ref[...] = jnp.dot(dis * jnp.maximum(t, 0.0), w_ref[...],
                       preferred_element_type=jnp.float32)


def _combine_body(a0_ref, a1_ref, y_ref, h0_ref, h1_ref, b_ref, o_ref):
  dis = _dis(h0_ref, h1_ref)
  o_ref[...] = dis * (a0_ref[...] + a1_ref[...] + y_ref[...]) + b_ref[...]


def _row_spec():
  return pl.BlockSpec((_R, D), lambda i: (i, 0))


def _col_spec():
  return pl.BlockSpec((_R, 1), lambda i: (i, 0))


def _full_spec(r):
  return pl.BlockSpec((r, D), lambda i: (0, 0))


_GRID = N_NODES // _R

_scale_matmul = pl.pallas_call(
    _scale_matmul_body,
    grid=(_GRID,),
    in_specs=[_row_spec(), _col_spec(), _col_spec(), _full_spec(D)],
    out_specs=_row_spec(),
    out_shape=jax.ShapeDtypeStruct((N_NODES, D), jnp.float32),
)

_combine_matmul = pl.pallas_call(
    _combine_matmul_body,
    grid=(_GRID,),
    in_specs=[_row_spec(), _row_spec(), _row_spec(), _col_spec(), _col_spec(),
              _full_spec(1), _full_spec(D)],
    out_specs=_row_spec(),
    out_shape=jax.ShapeDtypeStruct((N_NODES, D), jnp.float32),
)

_combine = pl.pallas_call(
    _combine_body,
    grid=(_GRID,),
    in_specs=[_row_spec(), _row_spec(), _row_spec(), _col_spec(), _col_spec(),
              _full_spec(1)],
    out_specs=_row_spec(),
    out_shape=jax.ShapeDtypeStruct((N_NODES, D), jnp.float32),
)


def kernel(x, edge_index, W1, b1, W2, b2):
  src = edge_index[0].astype(jnp.int32)
  dst = edge_index[1].astype(jnp.int32)
  npad = E_PAD - N_EDGES
  pad_src = jnp.arange(npad, dtype=jnp.int32) % N_NODES
  pad_dst = N_NODES + jnp.arange(npad, dtype=jnp.int32) % (ROWS_P - N_NODES)
  srcp = jnp.concatenate([src, pad_src]).reshape(NW, CPW, CHUNK)
  dstp = jnp.concatenate([dst, pad_dst]).reshape(NW, CPW, CHUNK)

  hist = _hist(dstp)
  h0 = hist[0, :N_NODES, 0:1]
  h1 = hist[1, :N_NODES, 0:1]

  zpad = jnp.zeros((ROWS_P - N_NODES, D), jnp.float32)

  y1 = _scale_matmul(x, h0, h1, W1)
  acc1 = _agg(jnp.concatenate([y1, zpad]), srcp, dstp)
  y2 = _combine_matmul(acc1[0, :N_NODES], acc1[1, :N_NODES], y1, h0, h1,
                       b1.reshape(1, D), W2)
  acc2 = _agg(jnp.concatenate([y2, zpad]), srcp, dstp)
  return _combine(acc2[0, :N_NODES], acc2[1, :N_NODES], y2, h0, h1,
                  b2.reshape(1, D))


# speedup vs baseline: 23.6480x; 1.0125x over previous
import functools

import jax
import jax.numpy as jnp
from jax import lax
from jax.experimental import pallas as pl
from jax.experimental.pallas import tpu as pltpu
from jax.experimental.pallas import tpu_sc as plsc

N_NODES = 10000
D = 128
N_EDGES = 320000

NC = 2
NS = 16
NW = NC * NS
CHUNK = 128
CPW = 80
E_PAD = NW * CPW * CHUNK
ROWS_P = 10240
RPT = ROWS_P // NS
HIST_W = D

_mesh = plsc.VectorSubcoreMesh(core_axis_name="c", subcore_axis_name="s")


def _zero_fill(buf, width):
  def body(i, _):
    for k in range(width // 16):
      buf[i, pl.ds(k * 16, 16)] = jnp.zeros((16,), jnp.float32)
    return 0
  lax.fori_loop(0, CHUNK, body, 0)


def _zero_acc_slice(zeros_v, acc_sh, s):
  base = s * RPT
  off = 0
  while off < RPT:
    n = min(CHUNK, RPT - off)
    pltpu.sync_copy(zeros_v.at[pl.ds(0, n)], acc_sh.at[pl.ds(base + off, n)])
    off += n


@functools.partial(
    pl.kernel,
    out_type=jax.ShapeDtypeStruct((NC, ROWS_P, HIST_W), jnp.float32),
    mesh=_mesh,
    scratch_types=[
        pltpu.VMEM((CPW, CHUNK), jnp.int32),
        pltpu.VMEM((CHUNK, HIST_W), jnp.float32),
        pltpu.VMEM((CHUNK, HIST_W), jnp.float32),
        pltpu.VMEM_SHARED((ROWS_P, HIST_W), jnp.float32),
    ],
)
def _hist(dst_hbm, out_hbm, idx_v, ones_v, zeros_v, acc_sh):
  c = lax.axis_index("c")
  s = lax.axis_index("s")
  w = s * NC + c

  def fill(i, _):
    for k in range(HIST_W // 16):
      ones_v[i, pl.ds(k * 16, 16)] = jnp.ones((16,), jnp.float32)
      zeros_v[i, pl.ds(k * 16, 16)] = jnp.zeros((16,), jnp.float32)
    return 0
  lax.fori_loop(0, CHUNK, fill, 0)

  _zero_acc_slice(zeros_v, acc_sh, s)
  plsc.subcore_barrier()

  pltpu.sync_copy(dst_hbm.at[w], idx_v)

  def body(j, _):
    pltpu.sync_copy(ones_v, acc_sh.at[idx_v.at[j]], add=True)
    return 0
  lax.fori_loop(0, CPW, body, 0)

  plsc.subcore_barrier()
  base = s * RPT
  pltpu.sync_copy(acc_sh.at[pl.ds(base, RPT)], out_hbm.at[c, pl.ds(base, RPT)])


NBUF = 2
SEG = 16


@functools.partial(
    pl.kernel,
    out_type=jax.ShapeDtypeStruct((NC, ROWS_P, D), jnp.float32),
    mesh=_mesh,
    scratch_types=[
        pltpu.VMEM((SEG, CHUNK), jnp.int32),
        pltpu.VMEM((SEG, CHUNK), jnp.int32),
        pltpu.VMEM((NBUF, CHUNK, D), jnp.float32),
        pltpu.VMEM_SHARED((ROWS_P, D), jnp.float32),
        pltpu.SemaphoreType.DMA,
        pltpu.SemaphoreType.DMA,
        pltpu.SemaphoreType.DMA,
        pltpu.SemaphoreType.DMA,
    ],
)
def _agg(table_hbm, src_hbm, dst_hbm, out_hbm, src_v, dst_v, buf_v, acc_sh,
         sem00, sem01, sem10, sem11):
  sems = ((sem00, sem01), (sem10, sem11))
  HALF = CHUNK // 2
  c = lax.axis_index("c")
  s = lax.axis_index("s")
  w = s * NC + c

  _zero_fill(buf_v.at[0], D)
  _zero_acc_slice(buf_v.at[0], acc_sh, s)
  plsc.subcore_barrier()

  def gather_chunk(j, b):
    pltpu.async_copy(table_hbm.at[src_v.at[j, pl.ds(0, HALF)]],
                     buf_v.at[b, pl.ds(0, HALF)], sems[b][0])
    pltpu.async_copy(table_hbm.at[src_v.at[j, pl.ds(HALF, HALF)]],
                     buf_v.at[b, pl.ds(HALF, HALF)], sems[b][1])

  def wait_chunk(j, b):
    pltpu.make_async_copy(table_hbm.at[src_v.at[j, pl.ds(0, HALF)]],
                          buf_v.at[b, pl.ds(0, HALF)], sems[b][0]).wait()
    pltpu.make_async_copy(table_hbm.at[src_v.at[j, pl.ds(HALF, HALF)]],
                          buf_v.at[b, pl.ds(HALF, HALF)], sems[b][1]).wait()

  def seg_body(t, _):
    pltpu.sync_copy(src_hbm.at[w, pl.ds(t * SEG, SEG)], src_v)
    pltpu.sync_copy(dst_hbm.at[w, pl.ds(t * SEG, SEG)], dst_v)
    gather_chunk(0, 0)

    def body(g, _):
      for b in range(NBUF):
        j = g * NBUF + b
        wait_chunk(j, b)
        jn = j + 1

        @pl.when(jn < SEG)
        def _():
          gather_chunk(jn, (b + 1) % NBUF)

        pltpu.sync_copy(buf_v.at[b], acc_sh.at[dst_v.at[j]], add=True)
      return 0
    lax.fori_loop(0, SEG // NBUF, body, 0)
    return 0
  lax.fori_loop(0, CPW // SEG, seg_body, 0)

  plsc.subcore_barrier()
  base = s * RPT
  pltpu.sync_copy(acc_sh.at[pl.ds(base, RPT)], out_hbm.at[c, pl.ds(base, RPT)])


_R = 1000


def _dis(h0, h1):
  return lax.rsqrt(1.0 + h0[...] + h1[...])


def _scale_matmul_body(x_ref, h0_ref, h1_ref, w_ref, o_ref):
  dis = _dis(h0_ref, h1_ref)
  o_ref[...] = jnp.dot(x_ref[...] * dis, w_ref[...],
                       preferred_element_type=jnp.float32)


def _combine_matmul_body(a0_ref, a1_ref, y_ref, h0_ref, h1_ref, b_ref, w_ref,
                         o_ref):
  dis = _dis(h0_ref, h1_ref)
  t = dis * (a0_ref[...] + a1_ref[...] + y_ref[...]) + b_ref[...]
  o_ref[...] = jnp.dot(dis * jnp.maximum(t, 0.0), w_ref[...],
                       preferred_element_type=jnp.float32)


def _combine_body(a0_ref, a1_ref, y_ref, h0_ref, h1_ref, b_ref, o_ref):
  dis = _dis(h0_ref, h1_ref)
  o_ref[...] = dis * (a0_ref[...] + a1_ref[...] + y_ref[...]) + b_ref[...]


def _row_spec():
  return pl.BlockSpec((_R, D), lambda i: (i, 0))


def _col_spec():
  return pl.BlockSpec((_R, 1), lambda i: (i, 0))


def _full_spec(r):
  return pl.BlockSpec((r, D), lambda i: (0, 0))


_GRID = N_NODES // _R

_scale_matmul = pl.pallas_call(
    _scale_matmul_body,
    grid=(_GRID,),
    in_specs=[_row_spec(), _col_spec(), _col_spec(), _full_spec(D)],
    out_specs=_row_spec(),
    out_shape=jax.ShapeDtypeStruct((N_NODES, D), jnp.float32),
)

_combine_matmul = pl.pallas_call(
    _combine_matmul_body,
    grid=(_GRID,),
    in_specs=[_row_spec(), _row_spec(), _row_spec(), _col_spec(), _col_spec(),
              _full_spec(1), _full_spec(D)],
    out_specs=_row_spec(),
    out_shape=jax.ShapeDtypeStruct((N_NODES, D), jnp.float32),
)

_combine = pl.pallas_call(
    _combine_body,
    grid=(_GRID,),
    in_specs=[_row_spec(), _row_spec(), _row_spec(), _col_spec(), _col_spec(),
              _full_spec(1)],
    out_specs=_row_spec(),
    out_shape=jax.ShapeDtypeStruct((N_NODES, D), jnp.float32),
)


def kernel(x, edge_index, W1, b1, W2, b2):
  src = edge_index[0].astype(jnp.int32)
  dst = edge_index[1].astype(jnp.int32)
  npad = E_PAD - N_EDGES
  pad_src = jnp.arange(npad, dtype=jnp.int32) % N_NODES
  pad_dst = N_NODES + jnp.arange(npad, dtype=jnp.int32) % (ROWS_P - N_NODES)
  srcp = jnp.concatenate([src, pad_src]).reshape(NW, CPW, CHUNK)
  dstp = jnp.concatenate([dst, pad_dst]).reshape(NW, CPW, CHUNK)

  hist = _hist(dstp)
  h0 = hist[0, :N_NODES, 0:1]
  h1 = hist[1, :N_NODES, 0:1]

  zpad = jnp.zeros((ROWS_P - N_NODES, D), jnp.float32)

  y1 = _scale_matmul(x, h0, h1, W1)
  acc1 = _agg(jnp.concatenate([y1, zpad]), srcp, dstp)
  y2 = _combine_matmul(acc1[0, :N_NODES], acc1[1, :N_NODES], y1, h0, h1,
                       b1.reshape(1, D), W2)
  acc2 = _agg(jnp.concatenate([y2, zpad]), srcp, dstp)
  return _combine(acc2[0, :N_NODES], acc2[1, :N_NODES], y2, h0, h1,
                  b2.reshape(1, D))


# speedup vs baseline: 27.2697x; 1.1531x over previous
import functools

import jax
import jax.numpy as jnp
from jax import lax
from jax.experimental import pallas as pl
from jax.experimental.pallas import tpu as pltpu
from jax.experimental.pallas import tpu_sc as plsc

N_NODES = 10000
D = 128
N_EDGES = 320000

NC = 2
NS = 16
NW = NC * NS
CHUNK = 128
CPW = 80
E_PAD = NW * CPW * CHUNK
ROWS_P = 10240
RPT = ROWS_P // NS

_mesh = plsc.VectorSubcoreMesh(core_axis_name="c", subcore_axis_name="s")


def _zero_fill(buf, width):
  def body(i, _):
    for k in range(width // 16):
      buf[i, pl.ds(k * 16, 16)] = jnp.zeros((16,), jnp.float32)
    return 0
  lax.fori_loop(0, CHUNK, body, 0)


def _zero_acc_slice(zeros_v, acc_sh, s):
  base = s * RPT
  off = 0
  while off < RPT:
    n = min(CHUNK, RPT - off)
    pltpu.sync_copy(zeros_v.at[pl.ds(0, n)], acc_sh.at[pl.ds(base + off, n)])
    off += n


HR = ROWS_P // D


@functools.partial(
    pl.kernel,
    out_type=jax.ShapeDtypeStruct((NC, HR, D), jnp.float32),
    mesh=_mesh,
    scratch_types=[
        pltpu.VMEM((CPW, CHUNK), jnp.int32),
        pltpu.VMEM((HR, D), jnp.float32),
        pltpu.VMEM((HR,), jnp.int32),
        pltpu.VMEM_SHARED((HR, D), jnp.float32),
    ],
    compiler_params=pltpu.CompilerParams(needs_layout_passes=False),
)
def _hist(dst_hbm, out_hbm, idx_v, h_v, rowid_v, acc_sh):
  c = lax.axis_index("c")
  s = lax.axis_index("s")
  w = s * NC + c
  rpt = 8

  def zfill(i, _):
    for k in range(D // 16):
      h_v[i, pl.ds(k * 16, 16)] = jnp.zeros((16,), jnp.float32)
    return 0
  lax.fori_loop(0, HR, zfill, 0)

  def rfill(i, _):
    rowid_v[pl.ds(i * 16, 16)] = i * 16 + lax.iota(jnp.int32, 16)
    return 0
  lax.fori_loop(0, HR // 16, rfill, 0)

  @pl.when(s < HR // rpt)
  def _():
    pltpu.sync_copy(h_v.at[pl.ds(0, rpt)], acc_sh.at[pl.ds(s * rpt, rpt)])
  plsc.subcore_barrier()

  pltpu.sync_copy(dst_hbm.at[w], idx_v)
  ones = jnp.ones((16,), jnp.float32)

  def body(j, _):
    for k in range(CHUNK // 16):
      ix = idx_v[j, pl.ds(k * 16, 16)]
      plsc.addupdate_scatter(h_v, [lax.shift_right_logical(ix, 7),
                                   lax.bitwise_and(ix, 127)], ones)
    return 0
  lax.fori_loop(0, CPW, body, 0)

  pltpu.sync_copy(h_v, acc_sh.at[rowid_v], add=True)
  plsc.subcore_barrier()

  @pl.when(s < HR // rpt)
  def _():
    pltpu.sync_copy(acc_sh.at[pl.ds(s * rpt, rpt)],
                    out_hbm.at[c, pl.ds(s * rpt, rpt)])


NBUF = 2
SEG = 16


@functools.partial(
    pl.kernel,
    out_type=jax.ShapeDtypeStruct((NC, ROWS_P, D), jnp.float32),
    mesh=_mesh,
    scratch_types=[
        pltpu.VMEM((SEG, CHUNK), jnp.int32),
        pltpu.VMEM((SEG, CHUNK), jnp.int32),
        pltpu.VMEM((NBUF, CHUNK, D), jnp.float32),
        pltpu.VMEM_SHARED((ROWS_P, D), jnp.float32),
        pltpu.SemaphoreType.DMA,
        pltpu.SemaphoreType.DMA,
        pltpu.SemaphoreType.DMA,
        pltpu.SemaphoreType.DMA,
    ],
)
def _agg(table_hbm, src_hbm, dst_hbm, out_hbm, src_v, dst_v, buf_v, acc_sh,
         sem00, sem01, sem10, sem11):
  sems = ((sem00, sem01), (sem10, sem11))
  HALF = CHUNK // 2
  c = lax.axis_index("c")
  s = lax.axis_index("s")
  w = s * NC + c

  _zero_fill(buf_v.at[0], D)
  _zero_acc_slice(buf_v.at[0], acc_sh, s)
  plsc.subcore_barrier()

  def gather_chunk(j, b):
    pltpu.async_copy(table_hbm.at[src_v.at[j, pl.ds(0, HALF)]],
                     buf_v.at[b, pl.ds(0, HALF)], sems[b][0])
    pltpu.async_copy(table_hbm.at[src_v.at[j, pl.ds(HALF, HALF)]],
                     buf_v.at[b, pl.ds(HALF, HALF)], sems[b][1])

  def wait_chunk(j, b):
    pltpu.make_async_copy(table_hbm.at[src_v.at[j, pl.ds(0, HALF)]],
                          buf_v.at[b, pl.ds(0, HALF)], sems[b][0]).wait()
    pltpu.make_async_copy(table_hbm.at[src_v.at[j, pl.ds(HALF, HALF)]],
                          buf_v.at[b, pl.ds(HALF, HALF)], sems[b][1]).wait()

  def seg_body(t, _):
    pltpu.sync_copy(src_hbm.at[w, pl.ds(t * SEG, SEG)], src_v)
    pltpu.sync_copy(dst_hbm.at[w, pl.ds(t * SEG, SEG)], dst_v)
    gather_chunk(0, 0)

    def body(g, _):
      for b in range(NBUF):
        j = g * NBUF + b
        wait_chunk(j, b)
        jn = j + 1

        @pl.when(jn < SEG)
        def _():
          gather_chunk(jn, (b + 1) % NBUF)

        pltpu.sync_copy(buf_v.at[b], acc_sh.at[dst_v.at[j]], add=True)
      return 0
    lax.fori_loop(0, SEG // NBUF, body, 0)
    return 0
  lax.fori_loop(0, CPW // SEG, seg_body, 0)

  plsc.subcore_barrier()
  base = s * RPT
  pltpu.sync_copy(acc_sh.at[pl.ds(base, RPT)], out_hbm.at[c, pl.ds(base, RPT)])


_R = 1000


def _dis(h0, h1):
  return lax.rsqrt(1.0 + h0[...] + h1[...])


def _scale_matmul_body(x_ref, h0_ref, h1_ref, w_ref, o_ref):
  dis = _dis(h0_ref, h1_ref)
  o_ref[...] = jnp.dot(x_ref[...] * dis, w_ref[...],
                       preferred_element_type=jnp.float32)


def _combine_matmul_body(a0_ref, a1_ref, y_ref, h0_ref, h1_ref, b_ref, w_ref,
                         o_ref):
  dis = _dis(h0_ref, h1_ref)
  t = dis * (a0_ref[...] + a1_ref[...] + y_ref[...]) + b_ref[...]
  o_ref[...] = jnp.dot(dis * jnp.maximum(t, 0.0), w_ref[...],
                       preferred_element_type=jnp.float32)


def _combine_body(a0_ref, a1_ref, y_ref, h0_ref, h1_ref, b_ref, o_ref):
  dis = _dis(h0_ref, h1_ref)
  o_ref[...] = dis * (a0_ref[...] + a1_ref[...] + y_ref[...]) + b_ref[...]


def _row_spec():
  return pl.BlockSpec((_R, D), lambda i: (i, 0))


def _col_spec():
  return pl.BlockSpec((_R, 1), lambda i: (i, 0))


def _full_spec(r):
  return pl.BlockSpec((r, D), lambda i: (0, 0))


_GRID = N_NODES // _R

_scale_matmul = pl.pallas_call(
    _scale_matmul_body,
    grid=(_GRID,),
    in_specs=[_row_spec(), _col_spec(), _col_spec(), _full_spec(D)],
    out_specs=_row_spec(),
    out_shape=jax.ShapeDtypeStruct((N_NODES, D), jnp.float32),
)

_combine_matmul = pl.pallas_call(
    _combine_matmul_body,
    grid=(_GRID,),
    in_specs=[_row_spec(), _row_spec(), _row_spec(), _col_spec(), _col_spec(),
              _full_spec(1), _full_spec(D)],
    out_specs=_row_spec(),
    out_shape=jax.ShapeDtypeStruct((N_NODES, D), jnp.float32),
)

_combine = pl.pallas_call(
    _combine_body,
    grid=(_GRID,),
    in_specs=[_row_spec(), _row_spec(), _row_spec(), _col_spec(), _col_spec(),
              _full_spec(1)],
    out_specs=_row_spec(),
    out_shape=jax.ShapeDtypeStruct((N_NODES, D), jnp.float32),
)


def kernel(x, edge_index, W1, b1, W2, b2):
  src = edge_index[0].astype(jnp.int32)
  dst = edge_index[1].astype(jnp.int32)
  npad = E_PAD - N_EDGES
  pad_src = jnp.arange(npad, dtype=jnp.int32) % N_NODES
  pad_dst = N_NODES + jnp.arange(npad, dtype=jnp.int32) % (ROWS_P - N_NODES)
  srcp = jnp.concatenate([src, pad_src]).reshape(NW, CPW, CHUNK)
  dstp = jnp.concatenate([dst, pad_dst]).reshape(NW, CPW, CHUNK)

  hist = _hist(dstp)
  h0 = hist[0].reshape(ROWS_P)[:N_NODES, None]
  h1 = hist[1].reshape(ROWS_P)[:N_NODES, None]

  zpad = jnp.zeros((ROWS_P - N_NODES, D), jnp.float32)

  y1 = _scale_matmul(x, h0, h1, W1)
  acc1 = _agg(jnp.concatenate([y1, zpad]), srcp, dstp)
  y2 = _combine_matmul(acc1[0, :N_NODES], acc1[1, :N_NODES], y1, h0, h1,
                       b1.reshape(1, D), W2)
  acc2 = _agg(jnp.concatenate([y2, zpad]), srcp, dstp)
  return _combine(acc2[0, :N_NODES], acc2[1, :N_NODES], y2, h0, h1,
                  b2.reshape(1, D))


# speedup vs baseline: 28.2594x; 1.0363x over previous
import functools

import jax
import jax.numpy as jnp
from jax import lax
from jax.experimental import pallas as pl
from jax.experimental.pallas import tpu as pltpu
from jax.experimental.pallas import tpu_sc as plsc

N_NODES = 10000
D = 128
N_EDGES = 320000

NC = 2
NS = 16
NW = NC * NS
CHUNK = 128
CPW = 80
E_PAD = NW * CPW * CHUNK
ROWS_P = 10240
RPT = ROWS_P // NS

_mesh = plsc.VectorSubcoreMesh(core_axis_name="c", subcore_axis_name="s")


def _zero_fill(buf, width):
  def body(i, _):
    for k in range(width // 16):
      buf[i, pl.ds(k * 16, 16)] = jnp.zeros((16,), jnp.float32)
    return 0
  lax.fori_loop(0, CHUNK, body, 0)


def _zero_acc_slice(zeros_v, acc_sh, s):
  base = s * RPT
  off = 0
  while off < RPT:
    n = min(CHUNK, RPT - off)
    pltpu.sync_copy(zeros_v.at[pl.ds(0, n)], acc_sh.at[pl.ds(base + off, n)])
    off += n


HR = ROWS_P // D


@functools.partial(
    pl.kernel,
    out_type=jax.ShapeDtypeStruct((NC, HR, D), jnp.float32),
    mesh=_mesh,
    scratch_types=[
        pltpu.VMEM((CPW, CHUNK), jnp.int32),
        pltpu.VMEM((HR, D), jnp.float32),
        pltpu.VMEM((HR,), jnp.int32),
        pltpu.VMEM_SHARED((HR, D), jnp.float32),
    ],
    compiler_params=pltpu.CompilerParams(needs_layout_passes=False),
)
def _hist(dst_hbm, out_hbm, idx_v, h_v, rowid_v, acc_sh):
  c = lax.axis_index("c")
  s = lax.axis_index("s")
  w = s * NC + c
  rpt = 8

  def zfill(i, _):
    for k in range(D // 16):
      h_v[i, pl.ds(k * 16, 16)] = jnp.zeros((16,), jnp.float32)
    return 0
  lax.fori_loop(0, HR, zfill, 0)

  def rfill(i, _):
    rowid_v[pl.ds(i * 16, 16)] = i * 16 + lax.iota(jnp.int32, 16)
    return 0
  lax.fori_loop(0, HR // 16, rfill, 0)

  @pl.when(s < HR // rpt)
  def _():
    pltpu.sync_copy(h_v.at[pl.ds(0, rpt)], acc_sh.at[pl.ds(s * rpt, rpt)])
  plsc.subcore_barrier()

  pltpu.sync_copy(dst_hbm.at[w], idx_v)
  ones = jnp.ones((16,), jnp.float32)

  def body(j, _):
    for k in range(CHUNK // 16):
      ix = idx_v[j, pl.ds(k * 16, 16)]
      plsc.addupdate_scatter(h_v, [lax.shift_right_logical(ix, 7),
                                   lax.bitwise_and(ix, 127)], ones)
    return 0
  lax.fori_loop(0, CPW, body, 0)

  pltpu.sync_copy(h_v, acc_sh.at[rowid_v], add=True)
  plsc.subcore_barrier()

  @pl.when(s < HR // rpt)
  def _():
    pltpu.sync_copy(acc_sh.at[pl.ds(s * rpt, rpt)],
                    out_hbm.at[c, pl.ds(s * rpt, rpt)])


NBUF = 2
SEG = 16


@functools.partial(
    pl.kernel,
    out_type=jax.ShapeDtypeStruct((NC, ROWS_P, D), jnp.float32),
    mesh=_mesh,
    scratch_types=[
        pltpu.VMEM((SEG, CHUNK), jnp.int32),
        pltpu.VMEM((SEG, CHUNK), jnp.int32),
        pltpu.VMEM((NBUF, CHUNK, D), jnp.float32),
        pltpu.VMEM_SHARED((ROWS_P, D), jnp.float32),
        pltpu.SemaphoreType.DMA,
        pltpu.SemaphoreType.DMA,
        pltpu.SemaphoreType.DMA,
        pltpu.SemaphoreType.DMA,
    ],
)
def _agg(table_hbm, src_hbm, dst_hbm, out_hbm, src_v, dst_v, buf_v, acc_sh,
         sem00, sem01, sem10, sem11):
  sems = ((sem00, sem01), (sem10, sem11))
  HALF = CHUNK // 2
  c = lax.axis_index("c")
  s = lax.axis_index("s")
  w = s * NC + c

  _zero_fill(buf_v.at[0], D)
  _zero_acc_slice(buf_v.at[0], acc_sh, s)
  plsc.subcore_barrier()

  def gather_chunk(j, b):
    pltpu.async_copy(table_hbm.at[src_v.at[j, pl.ds(0, HALF)]],
                     buf_v.at[b, pl.ds(0, HALF)], sems[b][0])
    pltpu.async_copy(table_hbm.at[src_v.at[j, pl.ds(HALF, HALF)]],
                     buf_v.at[b, pl.ds(HALF, HALF)], sems[b][1])

  def wait_chunk(j, b):
    pltpu.make_async_copy(table_hbm.at[src_v.at[j, pl.ds(0, HALF)]],
                          buf_v.at[b, pl.ds(0, HALF)], sems[b][0]).wait()
    pltpu.make_async_copy(table_hbm.at[src_v.at[j, pl.ds(HALF, HALF)]],
                          buf_v.at[b, pl.ds(HALF, HALF)], sems[b][1]).wait()

  def seg_body(t, _):
    pltpu.sync_copy(src_hbm.at[w, pl.ds(t * SEG, SEG)], src_v)
    pltpu.sync_copy(dst_hbm.at[w, pl.ds(t * SEG, SEG)], dst_v)
    gather_chunk(0, 0)

    def body(g, _):
      for b in range(NBUF):
        j = g * NBUF + b
        wait_chunk(j, b)
        jn = j + 1

        @pl.when(jn < SEG)
        def _():
          gather_chunk(jn, (b + 1) % NBUF)

        pltpu.sync_copy(buf_v.at[b], acc_sh.at[dst_v.at[j]], add=True)
      return 0
    lax.fori_loop(0, SEG // NBUF, body, 0)
    return 0
  lax.fori_loop(0, CPW // SEG, seg_body, 0)

  plsc.subcore_barrier()
  base = s * RPT
  pltpu.sync_copy(acc_sh.at[pl.ds(base, RPT)], out_hbm.at[c, pl.ds(base, RPT)])


_R = 2000


def _dis(h0, h1):
  return lax.rsqrt(1.0 + h0[...] + h1[...])


def _scale_matmul_body(x_ref, h0_ref, h1_ref, w_ref, o_ref):
  dis = _dis(h0_ref, h1_ref)
  o_ref[...] = jnp.dot(x_ref[...] * dis, w_ref[...],
                       preferred_element_type=jnp.float32)


def _combine_matmul_body(a0_ref, a1_ref, y_ref, h0_ref, h1_ref, b_ref, w_ref,
                         o_ref):
  dis = _dis(h0_ref, h1_ref)
  t = dis * (a0_ref[...] + a1_ref[...] + y_ref[...]) + b_ref[...]
  o_ref[...] = jnp.dot(dis * jnp.maximum(t, 0.0), w_ref[...],
                       preferred_element_type=jnp.float32)


def _combine_body(a0_ref, a1_ref, y_ref, h0_ref, h1_ref, b_ref, o_ref):
  dis = _dis(h0_ref, h1_ref)
  o_ref[...] = dis * (a0_ref[...] + a1_ref[...] + y_ref[...]) + b_ref[...]


def _row_spec():
  return pl.BlockSpec((_R, D), lambda i: (i, 0))


def _col_spec():
  return pl.BlockSpec((_R, 1), lambda i: (i, 0))


def _full_spec(r):
  return pl.BlockSpec((r, D), lambda i: (0, 0))


_GRID = N_NODES // _R

_scale_matmul = pl.pallas_call(
    _scale_matmul_body,
    grid=(_GRID,),
    in_specs=[_row_spec(), _col_spec(), _col_spec(), _full_spec(D)],
    out_specs=_row_spec(),
    out_shape=jax.ShapeDtypeStruct((N_NODES, D), jnp.float32),
)

_combine_matmul = pl.pallas_call(
    _combine_matmul_body,
    grid=(_GRID,),
    in_specs=[_row_spec(), _row_spec(), _row_spec(), _col_spec(), _col_spec(),
              _full_spec(1), _full_spec(D)],
    out_specs=_row_spec(),
    out_shape=jax.ShapeDtypeStruct((N_NODES, D), jnp.float32),
)

_combine = pl.pallas_call(
    _combine_body,
    grid=(_GRID,),
    in_specs=[_row_spec(), _row_spec(), _row_spec(), _col_spec(), _col_spec(),
              _full_spec(1)],
    out_specs=_row_spec(),
    out_shape=jax.ShapeDtypeStruct((N_NODES, D), jnp.float32),
)


def kernel(x, edge_index, W1, b1, W2, b2):
  src = edge_index[0].astype(jnp.int32)
  dst = edge_index[1].astype(jnp.int32)
  npad = E_PAD - N_EDGES
  pad_src = jnp.arange(npad, dtype=jnp.int32) % N_NODES
  pad_dst = N_NODES + jnp.arange(npad, dtype=jnp.int32) % (ROWS_P - N_NODES)
  srcp = jnp.concatenate([src, pad_src]).reshape(NW, CPW, CHUNK)
  dstp = jnp.concatenate([dst, pad_dst]).reshape(NW, CPW, CHUNK)

  hist = _hist(dstp)
  h0 = hist[0].reshape(ROWS_P)[:N_NODES, None]
  h1 = hist[1].reshape(ROWS_P)[:N_NODES, None]

  y1 = _scale_matmul(x, h0, h1, W1)
  acc1 = _agg(y1, srcp, dstp)
  y2 = _combine_matmul(acc1[0, :N_NODES], acc1[1, :N_NODES], y1, h0, h1,
                       b1.reshape(1, D), W2)
  acc2 = _agg(y2, srcp, dstp)
  return _combine(acc2[0, :N_NODES], acc2[1, :N_NODES], y2, h0, h1,
                  b2.reshape(1, D))


# speedup vs baseline: 29.4082x; 1.0407x over previous
import functools

import jax
import jax.numpy as jnp
from jax import lax
from jax.experimental import pallas as pl
from jax.experimental.pallas import tpu as pltpu
from jax.experimental.pallas import tpu_sc as plsc

N_NODES = 10000
D = 128
N_EDGES = 320000

NC = 2
NS = 16
NW = NC * NS
CHUNK = 128
CPW = 80
E_PAD = NW * CPW * CHUNK
ROWS_P = 10240
RPT = ROWS_P // NS

_mesh = plsc.VectorSubcoreMesh(core_axis_name="c", subcore_axis_name="s")


def _zero_fill(buf, width):
  def body(i, _):
    for k in range(width // 16):
      buf[i, pl.ds(k * 16, 16)] = jnp.zeros((16,), jnp.float32)
    return 0
  lax.fori_loop(0, CHUNK, body, 0)


def _zero_acc_slice(zeros_v, acc_sh, s):
  base = s * RPT
  off = 0
  while off < RPT:
    n = min(CHUNK, RPT - off)
    pltpu.sync_copy(zeros_v.at[pl.ds(0, n)], acc_sh.at[pl.ds(base + off, n)])
    off += n


HR = ROWS_P // D


@functools.partial(
    pl.kernel,
    out_type=jax.ShapeDtypeStruct((NC, HR, D), jnp.float32),
    mesh=_mesh,
    scratch_types=[
        pltpu.VMEM((CPW, CHUNK), jnp.int32),
        pltpu.VMEM((HR, D), jnp.float32),
        pltpu.VMEM((HR,), jnp.int32),
        pltpu.VMEM_SHARED((HR, D), jnp.float32),
    ],
    compiler_params=pltpu.CompilerParams(needs_layout_passes=False),
)
def _hist(dst_hbm, out_hbm, idx_v, h_v, rowid_v, acc_sh):
  c = lax.axis_index("c")
  s = lax.axis_index("s")
  w = s * NC + c
  rpt = 8

  def zfill(i, _):
    for k in range(D // 16):
      h_v[i, pl.ds(k * 16, 16)] = jnp.zeros((16,), jnp.float32)
    return 0
  lax.fori_loop(0, HR, zfill, 0)

  def rfill(i, _):
    rowid_v[pl.ds(i * 16, 16)] = i * 16 + lax.iota(jnp.int32, 16)
    return 0
  lax.fori_loop(0, HR // 16, rfill, 0)

  @pl.when(s < HR // rpt)
  def _():
    pltpu.sync_copy(h_v.at[pl.ds(0, rpt)], acc_sh.at[pl.ds(s * rpt, rpt)])
  plsc.subcore_barrier()

  pltpu.sync_copy(dst_hbm.at[w], idx_v)
  ones = jnp.ones((16,), jnp.float32)

  def body(j, _):
    for k in range(CHUNK // 16):
      ix = idx_v[j, pl.ds(k * 16, 16)]
      plsc.addupdate_scatter(h_v, [lax.shift_right_logical(ix, 7),
                                   lax.bitwise_and(ix, 127)], ones)
    return 0
  lax.fori_loop(0, CPW, body, 0)

  pltpu.sync_copy(h_v, acc_sh.at[rowid_v], add=True)
  plsc.subcore_barrier()

  @pl.when(s < HR // rpt)
  def _():
    pltpu.sync_copy(acc_sh.at[pl.ds(s * rpt, rpt)],
                    out_hbm.at[c, pl.ds(s * rpt, rpt)])


NBUF = 2
SEG = 16


@functools.partial(
    pl.kernel,
    out_type=jax.ShapeDtypeStruct((NC, ROWS_P, D), jnp.float32),
    mesh=_mesh,
    scratch_types=[
        pltpu.VMEM((SEG, CHUNK), jnp.int32),
        pltpu.VMEM((SEG, CHUNK), jnp.int32),
        pltpu.VMEM((NBUF, CHUNK, D), jnp.float32),
        pltpu.VMEM_SHARED((ROWS_P, D), jnp.float32),
        pltpu.SemaphoreType.DMA,
        pltpu.SemaphoreType.DMA,
        pltpu.SemaphoreType.DMA,
        pltpu.SemaphoreType.DMA,
    ],
)
def _agg(table_hbm, src_hbm, dst_hbm, out_hbm, src_v, dst_v, buf_v, acc_sh,
         sem00, sem01, sem10, sem11):
  sems = ((sem00, sem01), (sem10, sem11))
  HALF = CHUNK // 2
  c = lax.axis_index("c")
  s = lax.axis_index("s")
  w = s * NC + c

  _zero_fill(buf_v.at[0], D)
  _zero_acc_slice(buf_v.at[0], acc_sh, s)
  plsc.subcore_barrier()

  def gather_chunk(j, b):
    pltpu.async_copy(table_hbm.at[src_v.at[j, pl.ds(0, HALF)]],
                     buf_v.at[b, pl.ds(0, HALF)], sems[b][0])
    pltpu.async_copy(table_hbm.at[src_v.at[j, pl.ds(HALF, HALF)]],
                     buf_v.at[b, pl.ds(HALF, HALF)], sems[b][1])

  def wait_chunk(j, b):
    pltpu.make_async_copy(table_hbm.at[src_v.at[j, pl.ds(0, HALF)]],
                          buf_v.at[b, pl.ds(0, HALF)], sems[b][0]).wait()
    pltpu.make_async_copy(table_hbm.at[src_v.at[j, pl.ds(HALF, HALF)]],
                          buf_v.at[b, pl.ds(HALF, HALF)], sems[b][1]).wait()

  def seg_body(t, _):
    pltpu.sync_copy(src_hbm.at[w, pl.ds(t * SEG, SEG)], src_v)
    pltpu.sync_copy(dst_hbm.at[w, pl.ds(t * SEG, SEG)], dst_v)
    gather_chunk(0, 0)

    def body(g, _):
      for b in range(NBUF):
        j = g * NBUF + b
        wait_chunk(j, b)
        jn = j + 1

        @pl.when(jn < SEG)
        def _():
          gather_chunk(jn, (b + 1) % NBUF)

        pltpu.sync_copy(buf_v.at[b], acc_sh.at[dst_v.at[j]], add=True)
      return 0
    lax.fori_loop(0, SEG // NBUF, body, 0)
    return 0
  lax.fori_loop(0, CPW // SEG, seg_body, 0)

  plsc.subcore_barrier()
  base = s * RPT
  pltpu.sync_copy(acc_sh.at[pl.ds(base, RPT)], out_hbm.at[c, pl.ds(base, RPT)])


_R = 2000


def _dis(h0, h1):
  return lax.rsqrt(1.0 + h0[...] + h1[...])


def _scale_matmul_body(x_ref, h0_ref, h1_ref, w_ref, o_ref):
  dis = _dis(h0_ref, h1_ref)
  o_ref[...] = jnp.dot(x_ref[...] * dis, w_ref[...],
                       preferred_element_type=jnp.float32)


def _combine_matmul_body(a0_ref, a1_ref, y_ref, h0_ref, h1_ref, b_ref, w_ref,
                         o_ref):
  dis = _dis(h0_ref, h1_ref)
  t = dis * (a0_ref[0] + a1_ref[0] + y_ref[...]) + b_ref[...]
  o_ref[...] = jnp.dot(dis * jnp.maximum(t, 0.0), w_ref[...],
                       preferred_element_type=jnp.float32)


def _combine_body(a0_ref, a1_ref, y_ref, h0_ref, h1_ref, b_ref, o_ref):
  dis = _dis(h0_ref, h1_ref)
  o_ref[...] = dis * (a0_ref[0] + a1_ref[0] + y_ref[...]) + b_ref[...]


def _row_spec():
  return pl.BlockSpec((_R, D), lambda i: (i, 0))


def _acc_spec(plane):
  return pl.BlockSpec((1, _R, D), lambda i, p=plane: (p, i, 0))


def _col_spec():
  return pl.BlockSpec((_R, 1), lambda i: (i, 0))


def _full_spec(r):
  return pl.BlockSpec((r, D), lambda i: (0, 0))


_GRID = N_NODES // _R

_scale_matmul = pl.pallas_call(
    _scale_matmul_body,
    grid=(_GRID,),
    in_specs=[_row_spec(), _col_spec(), _col_spec(), _full_spec(D)],
    out_specs=_row_spec(),
    out_shape=jax.ShapeDtypeStruct((N_NODES, D), jnp.float32),
)

_combine_matmul = pl.pallas_call(
    _combine_matmul_body,
    grid=(_GRID,),
    in_specs=[_acc_spec(0), _acc_spec(1), _row_spec(), _col_spec(),
              _col_spec(), _full_spec(1), _full_spec(D)],
    out_specs=_row_spec(),
    out_shape=jax.ShapeDtypeStruct((N_NODES, D), jnp.float32),
)

_combine = pl.pallas_call(
    _combine_body,
    grid=(_GRID,),
    in_specs=[_acc_spec(0), _acc_spec(1), _row_spec(), _col_spec(),
              _col_spec(), _full_spec(1)],
    out_specs=_row_spec(),
    out_shape=jax.ShapeDtypeStruct((N_NODES, D), jnp.float32),
)


def kernel(x, edge_index, W1, b1, W2, b2):
  src = edge_index[0].astype(jnp.int32)
  dst = edge_index[1].astype(jnp.int32)
  npad = E_PAD - N_EDGES
  pad_src = jnp.arange(npad, dtype=jnp.int32) % N_NODES
  pad_dst = N_NODES + jnp.arange(npad, dtype=jnp.int32) % (ROWS_P - N_NODES)
  srcp = jnp.concatenate([src, pad_src]).reshape(NW, CPW, CHUNK)
  dstp = jnp.concatenate([dst, pad_dst]).reshape(NW, CPW, CHUNK)

  hist = _hist(dstp)
  h0 = hist[0].reshape(ROWS_P)[:N_NODES, None]
  h1 = hist[1].reshape(ROWS_P)[:N_NODES, None]

  y1 = _scale_matmul(x, h0, h1, W1)
  acc1 = _agg(y1, srcp, dstp)
  y2 = _combine_matmul(acc1, acc1, y1, h0, h1, b1.reshape(1, D), W2)
  acc2 = _agg(y2, srcp, dstp)
  return _combine(acc2, acc2, y2, h0, h1, b2.reshape(1, D))


# speedup vs baseline: 31.1173x; 1.0581x over previous
import functools

import jax
import jax.numpy as jnp
from jax import lax
from jax.experimental import pallas as pl
from jax.experimental.pallas import tpu as pltpu
from jax.experimental.pallas import tpu_sc as plsc

N_NODES = 10000
D = 128
N_EDGES = 320000

NC = 2
NS = 16
NW = NC * NS
CHUNK = 128
CPW = 80
E_PAD = NW * CPW * CHUNK
ROWS_P = 10240
RPT = ROWS_P // NS

_mesh = plsc.VectorSubcoreMesh(core_axis_name="c", subcore_axis_name="s")


def _zero_fill(buf, width):
  def body(i, _):
    for k in range(width // 16):
      buf[i, pl.ds(k * 16, 16)] = jnp.zeros((16,), jnp.float32)
    return 0
  lax.fori_loop(0, CHUNK, body, 0)


def _zero_acc_slice(zeros_v, acc_sh, s):
  base = s * RPT
  off = 0
  while off < RPT:
    n = min(CHUNK, RPT - off)
    pltpu.sync_copy(zeros_v.at[pl.ds(0, n)], acc_sh.at[pl.ds(base + off, n)])
    off += n


HR = ROWS_P // D


@functools.partial(
    pl.kernel,
    out_type=jax.ShapeDtypeStruct((NC, HR, D), jnp.float32),
    mesh=_mesh,
    scratch_types=[
        pltpu.VMEM((CPW, CHUNK), jnp.int32),
        pltpu.VMEM((HR, D), jnp.float32),
        pltpu.VMEM((HR,), jnp.int32),
        pltpu.VMEM_SHARED((HR, D), jnp.float32),
    ],
    compiler_params=pltpu.CompilerParams(needs_layout_passes=False),
)
def _hist(dst_hbm, out_hbm, idx_v, h_v, rowid_v, acc_sh):
  c = lax.axis_index("c")
  s = lax.axis_index("s")
  w = s * NC + c
  rpt = 8

  def zfill(i, _):
    for k in range(D // 16):
      h_v[i, pl.ds(k * 16, 16)] = jnp.zeros((16,), jnp.float32)
    return 0
  lax.fori_loop(0, HR, zfill, 0)

  def rfill(i, _):
    rowid_v[pl.ds(i * 16, 16)] = i * 16 + lax.iota(jnp.int32, 16)
    return 0
  lax.fori_loop(0, HR // 16, rfill, 0)

  @pl.when(s < HR // rpt)
  def _():
    pltpu.sync_copy(h_v.at[pl.ds(0, rpt)], acc_sh.at[pl.ds(s * rpt, rpt)])
  plsc.subcore_barrier()

  pltpu.sync_copy(dst_hbm.at[w], idx_v)
  ones = jnp.ones((16,), jnp.float32)

  def body(j, _):
    for k in range(CHUNK // 16):
      ix = idx_v[j, pl.ds(k * 16, 16)]
      plsc.addupdate_scatter(h_v, [lax.shift_right_logical(ix, 7),
                                   lax.bitwise_and(ix, 127)], ones)
    return 0
  lax.fori_loop(0, CPW, body, 0)

  pltpu.sync_copy(h_v, acc_sh.at[rowid_v], add=True)
  plsc.subcore_barrier()

  @pl.when(s < HR // rpt)
  def _():
    pltpu.sync_copy(acc_sh.at[pl.ds(s * rpt, rpt)],
                    out_hbm.at[c, pl.ds(s * rpt, rpt)])


NBUF = 2
SEGC = 8
PAIR = 2 * SEGC
NP = CPW // PAIR


@functools.partial(
    pl.kernel,
    out_type=jax.ShapeDtypeStruct((NC, ROWS_P, D), jnp.float32),
    mesh=_mesh,
    scratch_types=[
        pltpu.VMEM((SEGC, CHUNK), jnp.int32),
        pltpu.VMEM((SEGC, CHUNK), jnp.int32),
        pltpu.VMEM((SEGC, CHUNK), jnp.int32),
        pltpu.VMEM((SEGC, CHUNK), jnp.int32),
        pltpu.VMEM((NBUF, CHUNK, D), jnp.float32),
        pltpu.VMEM_SHARED((ROWS_P, D), jnp.float32),
        pltpu.SemaphoreType.DMA,
        pltpu.SemaphoreType.DMA,
        pltpu.SemaphoreType.DMA,
        pltpu.SemaphoreType.DMA,
        pltpu.SemaphoreType.DMA,
        pltpu.SemaphoreType.DMA,
    ],
)
def _agg(table_hbm, src_hbm, dst_hbm, out_hbm, src_a, dst_a, src_b, dst_b,
         buf_v, acc_sh, semg00, semg01, semg10, semg11, sem_ia, sem_ib):
  gsems = ((semg00, semg01), (semg10, semg11))
  HALF = CHUNK // 2
  c = lax.axis_index("c")
  s = lax.axis_index("s")
  w = s * NC + c

  def gather_chunk(idx_ref, r, b):
    pltpu.async_copy(table_hbm.at[idx_ref.at[r, pl.ds(0, HALF)]],
                     buf_v.at[b, pl.ds(0, HALF)], gsems[b][0])
    pltpu.async_copy(table_hbm.at[idx_ref.at[r, pl.ds(HALF, HALF)]],
                     buf_v.at[b, pl.ds(HALF, HALF)], gsems[b][1])

  def wait_chunk(idx_ref, r, b):
    pltpu.make_async_copy(table_hbm.at[idx_ref.at[r, pl.ds(0, HALF)]],
                          buf_v.at[b, pl.ds(0, HALF)], gsems[b][0]).wait()
    pltpu.make_async_copy(table_hbm.at[idx_ref.at[r, pl.ds(HALF, HALF)]],
                          buf_v.at[b, pl.ds(HALF, HALF)], gsems[b][1]).wait()

  def stage(seg, src_ref, dst_ref, sem):
    pltpu.async_copy(src_hbm.at[w, pl.ds(seg * SEGC, SEGC)], src_ref, sem)
    pltpu.async_copy(dst_hbm.at[w, pl.ds(seg * SEGC, SEGC)], dst_ref, sem)

  def stage_wait(seg, src_ref, dst_ref, sem):
    pltpu.make_async_copy(src_hbm.at[w, pl.ds(seg * SEGC, SEGC)], src_ref,
                          sem).wait()
    pltpu.make_async_copy(dst_hbm.at[w, pl.ds(seg * SEGC, SEGC)], dst_ref,
                          sem).wait()

  pltpu.sync_copy(src_hbm.at[w, pl.ds(0, SEGC)], src_a)
  pltpu.sync_copy(dst_hbm.at[w, pl.ds(0, SEGC)], dst_a)
  stage(1, src_b, dst_b, sem_ib)
  gather_chunk(src_a, 0, 0)
  _zero_fill(buf_v.at[1], D)
  _zero_acc_slice(buf_v.at[1], acc_sh, s)
  plsc.subcore_barrier()

  def body(t, _):
    for jl in range(PAIR):
      side_a = jl < SEGC
      idx_s = src_a if side_a else src_b
      idx_d = dst_a if side_a else dst_b
      r = jl % SEGC
      b = jl % NBUF

      if jl == SEGC - 1:
        stage_wait(2 * t + 1, src_b, dst_b, sem_ib)

      wait_chunk(idx_s, r, b)

      jn = jl + 1
      if jn < PAIR:
        gather_chunk(src_a if jn < SEGC else src_b, jn % SEGC, (b + 1) % NBUF)
      else:
        @pl.when(t < NP - 1)
        def _():
          stage_wait(2 * t + 2, src_a, dst_a, sem_ia)
          gather_chunk(src_a, 0, (b + 1) % NBUF)

      pltpu.sync_copy(buf_v.at[b], acc_sh.at[idx_d.at[r]], add=True)

      if jl == SEGC:
        @pl.when(t < NP - 1)
        def _():
          stage(2 * t + 2, src_a, dst_a, sem_ia)
      if jl == PAIR - 1:
        @pl.when(t < NP - 1)
        def _():
          stage(2 * t + 3, src_b, dst_b, sem_ib)
    return 0
  lax.fori_loop(0, NP, body, 0)

  plsc.subcore_barrier()
  base = s * RPT
  pltpu.sync_copy(acc_sh.at[pl.ds(base, RPT)], out_hbm.at[c, pl.ds(base, RPT)])


_R = 2000


def _dis(h0, h1):
  return lax.rsqrt(1.0 + h0[...] + h1[...])


def _scale_matmul_body(x_ref, h0_ref, h1_ref, w_ref, o_ref):
  dis = _dis(h0_ref, h1_ref)
  o_ref[...] = jnp.dot(x_ref[...] * dis, w_ref[...],
                       preferred_element_type=jnp.float32)


def _combine_matmul_body(a0_ref, a1_ref, y_ref, h0_ref, h1_ref, b_ref, w_ref,
                         o_ref):
  dis = _dis(h0_ref, h1_ref)
  t = dis * (a0_ref[0] + a1_ref[0] + y_ref[...]) + b_ref[...]
  o_ref[...] = jnp.dot(dis * jnp.maximum(t, 0.0), w_ref[...],
                       preferred_element_type=jnp.float32)


def _combine_body(a0_ref, a1_ref, y_ref, h0_ref, h1_ref, b_ref, o_ref):
  dis = _dis(h0_ref, h1_ref)
  o_ref[...] = dis * (a0_ref[0] + a1_ref[0] + y_ref[...]) + b_ref[...]


def _row_spec():
  return pl.BlockSpec((_R, D), lambda i: (i, 0))


def _acc_spec(plane):
  return pl.BlockSpec((1, _R, D), lambda i, p=plane: (p, i, 0))


def _col_spec():
  return pl.BlockSpec((_R, 1), lambda i: (i, 0))


def _full_spec(r):
  return pl.BlockSpec((r, D), lambda i: (0, 0))


_GRID = N_NODES // _R

_scale_matmul = pl.pallas_call(
    _scale_matmul_body,
    grid=(_GRID,),
    in_specs=[_row_spec(), _col_spec(), _col_spec(), _full_spec(D)],
    out_specs=_row_spec(),
    out_shape=jax.ShapeDtypeStruct((N_NODES, D), jnp.float32),
)

_combine_matmul = pl.pallas_call(
    _combine_matmul_body,
    grid=(_GRID,),
    in_specs=[_acc_spec(0), _acc_spec(1), _row_spec(), _col_spec(),
              _col_spec(), _full_spec(1), _full_spec(D)],
    out_specs=_row_spec(),
    out_shape=jax.ShapeDtypeStruct((N_NODES, D), jnp.float32),
)

_combine = pl.pallas_call(
    _combine_body,
    grid=(_GRID,),
    in_specs=[_acc_spec(0), _acc_spec(1), _row_spec(), _col_spec(),
              _col_spec(), _full_spec(1)],
    out_specs=_row_spec(),
    out_shape=jax.ShapeDtypeStruct((N_NODES, D), jnp.float32),
)


def kernel(x, edge_index, W1, b1, W2, b2):
  src = edge_index[0].astype(jnp.int32)
  dst = edge_index[1].astype(jnp.int32)
  npad = E_PAD - N_EDGES
  pad_src = jnp.arange(npad, dtype=jnp.int32) % N_NODES
  pad_dst = N_NODES + jnp.arange(npad, dtype=jnp.int32) % (ROWS_P - N_NODES)
  srcp = jnp.concatenate([src, pad_src]).reshape(NW, CPW, CHUNK)
  dstp = jnp.concatenate([dst, pad_dst]).reshape(NW, CPW, CHUNK)

  hist = _hist(dstp)
  h0 = hist[0].reshape(ROWS_P)[:N_NODES, None]
  h1 = hist[1].reshape(ROWS_P)[:N_NODES, None]

  y1 = _scale_matmul(x, h0, h1, W1)
  acc1 = _agg(y1, srcp, dstp)
  y2 = _combine_matmul(acc1, acc1, y1, h0, h1, b1.reshape(1, D), W2)
  acc2 = _agg(y2, srcp, dstp)
  return _combine(acc2, acc2, y2, h0, h1, b2.reshape(1, D))


# speedup vs baseline: 38.8933x; 1.2499x over previous
import functools

import jax
import jax.numpy as jnp
from jax import lax
from jax.experimental import pallas as pl
from jax.experimental.pallas import tpu as pltpu
from jax.experimental.pallas import tpu_sc as plsc

N_NODES = 10000
D = 128
N_EDGES = 320000

NC = 2
NS = 16
NW = NC * NS
CHUNK = 80
CPW = 128
E_PAD = NW * CPW * CHUNK
ROWS_P = 10240
RPT = ROWS_P // NS

_mesh = plsc.VectorSubcoreMesh(core_axis_name="c", subcore_axis_name="s")


def _zero_fill(buf, width):
  def body(i, _):
    for k in range(width // 16):
      buf[i, pl.ds(k * 16, 16)] = jnp.zeros((16,), jnp.float32)
    return 0
  lax.fori_loop(0, CHUNK, body, 0)


def _zero_acc_slice(zeros_v, acc_sh, s):
  base = s * RPT
  off = 0
  while off < RPT:
    n = min(CHUNK, RPT - off)
    pltpu.sync_copy(zeros_v.at[pl.ds(0, n)], acc_sh.at[pl.ds(base + off, n)])
    off += n


HR = ROWS_P // D


@functools.partial(
    pl.kernel,
    out_type=jax.ShapeDtypeStruct((NC, HR, D), jnp.float32),
    mesh=_mesh,
    scratch_types=[
        pltpu.VMEM((CPW, CHUNK), jnp.int32),
        pltpu.VMEM((HR, D), jnp.float32),
        pltpu.VMEM((HR,), jnp.int32),
        pltpu.VMEM_SHARED((HR, D), jnp.float32),
    ],
    compiler_params=pltpu.CompilerParams(needs_layout_passes=False),
)
def _hist(dst_hbm, out_hbm, idx_v, h_v, rowid_v, acc_sh):
  c = lax.axis_index("c")
  s = lax.axis_index("s")
  w = s * NC + c
  rpt = 8

  def zfill(i, _):
    for k in range(D // 16):
      h_v[i, pl.ds(k * 16, 16)] = jnp.zeros((16,), jnp.float32)
    return 0
  lax.fori_loop(0, HR, zfill, 0)

  def rfill(i, _):
    rowid_v[pl.ds(i * 16, 16)] = i * 16 + lax.iota(jnp.int32, 16)
    return 0
  lax.fori_loop(0, HR // 16, rfill, 0)

  @pl.when(s < HR // rpt)
  def _():
    pltpu.sync_copy(h_v.at[pl.ds(0, rpt)], acc_sh.at[pl.ds(s * rpt, rpt)])
  plsc.subcore_barrier()

  pltpu.sync_copy(dst_hbm.at[w], idx_v)
  ones = jnp.ones((16,), jnp.float32)

  def body(j, _):
    for k in range(CHUNK // 16):
      ix = idx_v[j, pl.ds(k * 16, 16)]
      plsc.addupdate_scatter(h_v, [lax.shift_right_logical(ix, 7),
                                   lax.bitwise_and(ix, 127)], ones)
    return 0
  lax.fori_loop(0, CPW, body, 0)

  pltpu.sync_copy(h_v, acc_sh.at[rowid_v], add=True)
  plsc.subcore_barrier()

  @pl.when(s < HR // rpt)
  def _():
    pltpu.sync_copy(acc_sh.at[pl.ds(s * rpt, rpt)],
                    out_hbm.at[c, pl.ds(s * rpt, rpt)])


NBUF = 4
DEPTH = NBUF - 1
SEGC = 8
PAIR = 2 * SEGC
NP = CPW // PAIR


@functools.partial(
    pl.kernel,
    out_type=jax.ShapeDtypeStruct((NC, ROWS_P, D), jnp.float32),
    mesh=_mesh,
    scratch_types=[
        pltpu.VMEM((SEGC, CHUNK), jnp.int32),
        pltpu.VMEM((SEGC, CHUNK), jnp.int32),
        pltpu.VMEM((SEGC, CHUNK), jnp.int32),
        pltpu.VMEM((SEGC, CHUNK), jnp.int32),
        pltpu.VMEM((NBUF, CHUNK, D), jnp.float32),
        pltpu.VMEM_SHARED((ROWS_P, D), jnp.float32),
        pltpu.SemaphoreType.DMA,
        pltpu.SemaphoreType.DMA,
        pltpu.SemaphoreType.DMA,
        pltpu.SemaphoreType.DMA,
        pltpu.SemaphoreType.DMA,
        pltpu.SemaphoreType.DMA,
    ],
)
def _agg(table_hbm, src_hbm, dst_hbm, out_hbm, src_a, dst_a, src_b, dst_b,
         buf_v, acc_sh, semg0, semg1, semg2, semg3, sem_ia, sem_ib):
  gsems = (semg0, semg1, semg2, semg3)
  c = lax.axis_index("c")
  s = lax.axis_index("s")
  w = s * NC + c

  def gather_chunk(idx_ref, r, b):
    pltpu.async_copy(table_hbm.at[idx_ref.at[r]], buf_v.at[b], gsems[b])

  def wait_chunk(idx_ref, r, b):
    pltpu.make_async_copy(table_hbm.at[idx_ref.at[r]], buf_v.at[b],
                          gsems[b]).wait()

  def stage(seg, src_ref, dst_ref, sem):
    pltpu.async_copy(src_hbm.at[w, pl.ds(seg * SEGC, SEGC)], src_ref, sem)
    pltpu.async_copy(dst_hbm.at[w, pl.ds(seg * SEGC, SEGC)], dst_ref, sem)

  def stage_wait(seg, src_ref, dst_ref, sem):
    pltpu.make_async_copy(src_hbm.at[w, pl.ds(seg * SEGC, SEGC)], src_ref,
                          sem).wait()
    pltpu.make_async_copy(dst_hbm.at[w, pl.ds(seg * SEGC, SEGC)], dst_ref,
                          sem).wait()

  pltpu.sync_copy(src_hbm.at[w, pl.ds(0, SEGC)], src_a)
  pltpu.sync_copy(dst_hbm.at[w, pl.ds(0, SEGC)], dst_a)
  stage(1, src_b, dst_b, sem_ib)
  for p in range(DEPTH):
    gather_chunk(src_a, p, p)
  _zero_fill(buf_v.at[DEPTH], D)
  _zero_acc_slice(buf_v.at[DEPTH], acc_sh, s)
  plsc.subcore_barrier()

  def body(t, _):
    for jl in range(PAIR):
      side_a = jl < SEGC
      idx_s = src_a if side_a else src_b
      idx_d = dst_a if side_a else dst_b
      r = jl % SEGC
      b = jl % NBUF

      if jl == SEGC - DEPTH:
        stage_wait(2 * t + 1, src_b, dst_b, sem_ib)

      wait_chunk(idx_s, r, b)

      jn = jl + DEPTH
      if jn < PAIR:
        gather_chunk(src_a if jn < SEGC else src_b, jn % SEGC, jn % NBUF)
      else:
        if jl == PAIR - DEPTH:
          @pl.when(t < NP - 1)
          def _():
            stage_wait(2 * t + 2, src_a, dst_a, sem_ia)

        @pl.when(t < NP - 1)
        def _():
          gather_chunk(src_a, jn % SEGC, jn % NBUF)

      pltpu.sync_copy(buf_v.at[b], acc_sh.at[idx_d.at[r]], add=True)

      if jl == SEGC:
        @pl.when(t < NP - 1)
        def _():
          stage(2 * t + 2, src_a, dst_a, sem_ia)
      if jl == PAIR - 1:
        @pl.when(t < NP - 1)
        def _():
          stage(2 * t + 3, src_b, dst_b, sem_ib)
    return 0
  lax.fori_loop(0, NP, body, 0)

  plsc.subcore_barrier()
  base = s * RPT
  pltpu.sync_copy(acc_sh.at[pl.ds(base, RPT)], out_hbm.at[c, pl.ds(base, RPT)])


_R = 2000


def _dis(h0, h1):
  return lax.rsqrt(1.0 + h0[...] + h1[...])


def _scale_matmul_body(x_ref, h0_ref, h1_ref, w_ref, o_ref):
  dis = _dis(h0_ref, h1_ref)
  o_ref[...] = jnp.dot(x_ref[...] * dis, w_ref[...],
                       preferred_element_type=jnp.float32)


def _combine_matmul_body(a0_ref, a1_ref, y_ref, h0_ref, h1_ref, b_ref, w_ref,
                         o_ref):
  dis = _dis(h0_ref, h1_ref)
  t = dis * (a0_ref[0] + a1_ref[0] + y_ref[...]) + b_ref[...]
  o_ref[...] = jnp.dot(dis * jnp.maximum(t, 0.0), w_ref[...],
                       preferred_element_type=jnp.float32)


def _combine_body(a0_ref, a1_ref, y_ref, h0_ref, h1_ref, b_ref, o_ref):
  dis = _dis(h0_ref, h1_ref)
  o_ref[...] = dis * (a0_ref[0] + a1_ref[0] + y_ref[...]) + b_ref[...]


def _row_spec():
  return pl.BlockSpec((_R, D), lambda i: (i, 0))


def _acc_spec(plane):
  return pl.BlockSpec((1, _R, D), lambda i, p=plane: (p, i, 0))


def _col_spec():
  return pl.BlockSpec((_R, 1), lambda i: (i, 0))


def _full_spec(r):
  return pl.BlockSpec((r, D), lambda i: (0, 0))


_GRID = N_NODES // _R

_scale_matmul = pl.pallas_call(
    _scale_matmul_body,
    grid=(_GRID,),
    in_specs=[_row_spec(), _col_spec(), _col_spec(), _full_spec(D)],
    out_specs=_row_spec(),
    out_shape=jax.ShapeDtypeStruct((N_NODES, D), jnp.float32),
)

_combine_matmul = pl.pallas_call(
    _combine_matmul_body,
    grid=(_GRID,),
    in_specs=[_acc_spec(0), _acc_spec(1), _row_spec(), _col_spec(),
              _col_spec(), _full_spec(1), _full_spec(D)],
    out_specs=_row_spec(),
    out_shape=jax.ShapeDtypeStruct((N_NODES, D), jnp.float32),
)

_combine = pl.pallas_call(
    _combine_body,
    grid=(_GRID,),
    in_specs=[_acc_spec(0), _acc_spec(1), _row_spec(), _col_spec(),
              _col_spec(), _full_spec(1)],
    out_specs=_row_spec(),
    out_shape=jax.ShapeDtypeStruct((N_NODES, D), jnp.float32),
)


def kernel(x, edge_index, W1, b1, W2, b2):
  src = edge_index[0].astype(jnp.int32)
  dst = edge_index[1].astype(jnp.int32)
  npad = E_PAD - N_EDGES
  pad_src = jnp.arange(npad, dtype=jnp.int32) % N_NODES
  pad_dst = N_NODES + jnp.arange(npad, dtype=jnp.int32) % (ROWS_P - N_NODES)
  srcp = jnp.concatenate([src, pad_src]).reshape(NW, CPW, CHUNK)
  dstp = jnp.concatenate([dst, pad_dst]).reshape(NW, CPW, CHUNK)

  hist = _hist(dstp)
  h0 = hist[0].reshape(ROWS_P)[:N_NODES, None]
  h1 = hist[1].reshape(ROWS_P)[:N_NODES, None]

  y1 = _scale_matmul(x, h0, h1, W1)
  acc1 = _agg(y1, srcp, dstp)
  y2 = _combine_matmul(acc1, acc1, y1, h0, h1, b1.reshape(1, D), W2)
  acc2 = _agg(y2, srcp, dstp)
  return _combine(acc2, acc2, y2, h0, h1, b2.reshape(1, D))
